# baseline (speedup 1.0000x reference)
import functools

import jax
import jax.numpy as jnp
from jax import lax
from jax.experimental import pallas as pl
from jax.experimental.pallas import tpu as pltpu
from jax.experimental.pallas import tpu_sc as plsc

ROWS = 128
COLS = 32768
K = 64
L = 16
BSHIFT = 21
NBUCKET = 1 << (31 - BSHIFT)
CAND_ROWS = 256
SIGN_MASK = 0x7FFFFFFF
NC = 2
NS = 16
ROWS_PER_W = ROWS // (NC * NS)


def _sc_body(in_hbm, out_hbm, row_a, row_b, hist_v, cand_v,
             sem_ai, sem_bi, sem_ao, sem_bo):
    wid = lax.axis_index("s") * NC + lax.axis_index("c")
    lane = lax.iota(jnp.int32, L)
    ones = jnp.ones((L,), jnp.int32)
    zeros = jnp.zeros((L,), jnp.int32)
    base_row = wid * ROWS_PER_W
    bufs = [(row_a, sem_ai, sem_ao), (row_b, sem_bi, sem_bo)]

    def process_row(row_v, row):
        def hist_loop(i, mx):
            v = plsc.bitcast(row_v[pl.ds(i, L)], jnp.int32)
            ab = v & SIGN_MASK
            idx = ((ab >> BSHIFT) << 4) | lane
            plsc.addupdate_scatter(hist_v, [idx], ones)
            return jnp.maximum(mx, ab)
        with jax.named_scope("hist"):
            mxv = plsc.parallel_loop(0, COLS, L, unroll=8,
                                     carry=zeros)(hist_loop)
        bstart = jnp.max(mxv) >> BSHIFT

        def scan_cond(st):
            b, cum, _ = st
            return jnp.logical_and(cum < K, b >= 0)

        def scan_body(st):
            b, cum, _ = st
            c = jnp.sum(hist_v[pl.ds(b * L, L)])
            return (b - 1, cum + c, c)
        with jax.named_scope("scan"):
            bf, cum, lastc = lax.while_loop(
                scan_cond, scan_body, (bstart, jnp.int32(0), jnp.int32(0)))
        bucket = bf + 1
        need = K - (cum - lastc)

        def collect_loop(i, cs):
            v = plsc.bitcast(row_v[pl.ds(i, L)], jnp.int32)
            ab = v & SIGN_MASK
            m = jnp.logical_and((ab >> BSHIFT) == bucket,
                                cs < CAND_ROWS * L)
            plsc.store_scatter(cand_v, [cs], ab, mask=m)
            return cs + jnp.where(m, L, 0)
        with jax.named_scope("collect"):
            cs = plsc.parallel_loop(0, COLS, L, unroll=8,
                                    carry=lane)(collect_loop)
        c_vec = (cs - lane) >> 4
        max_c = jnp.max(c_vec)

        base = bucket << BSHIFT

        def bs_loop(_, st):
            lo, hi = st
            mid = lo + ((hi - lo + 1) >> 1)

            def count_loop(j, acc):
                cv = cand_v[pl.ds(j * L, L)]
                pm = jnp.logical_and(cv >= mid, j < c_vec)
                return acc + jnp.where(pm, 1, 0)
            c = jnp.sum(lax.fori_loop(0, max_c, count_loop, zeros))
            ok = c >= need
            return (jnp.where(ok, mid, lo), jnp.where(ok, hi, mid - 1))
        with jax.named_scope("bsearch"):
            thr, _ = lax.fori_loop(0, 21, bs_loop,
                                   (base, base + (1 << BSHIFT) - 1))

        def tie_loop(j, acc):
            cv = cand_v[pl.ds(j * L, L)]
            valid = j < c_vec
            ge = jnp.logical_and(cv >= thr, valid)
            eq = jnp.logical_and(cv == thr, valid)
            return (acc[0] + jnp.where(ge, 1, 0),
                    acc[1] + jnp.where(eq, 1, 0))
        gev, eqv = lax.fori_loop(0, max_c, tie_loop, (zeros, zeros))
        n_ge = (K - need) + jnp.sum(gev)
        n_eq = jnp.sum(eqv)
        t_keep = K - (n_ge - n_eq)

        def mask_loop(i):
            v = plsc.bitcast(row_v[pl.ds(i, L)], jnp.int32)
            keep = (v & SIGN_MASK) >= thr
            row_v[pl.ds(i, L)] = plsc.bitcast(
                jnp.where(keep, v, 0), jnp.float32)
        with jax.named_scope("mask"):
            plsc.parallel_loop(0, COLS, L, unroll=8)(mask_loop)

        @pl.when(n_ge > K)
        def _fixup():
            def fx_cond(st):
                i, c = st
                return jnp.logical_and(i < COLS, c < n_eq)

            def fx_body(st):
                i, c = st
                v = plsc.bitcast(row_v[pl.ds(i, L)], jnp.int32)
                eqm = (v & SIGN_MASK) == thr
                rank = c + plsc.cumsum(jnp.where(eqm, 1, 0)) - 1
                drop = jnp.logical_and(eqm, rank >= t_keep)
                row_v[pl.ds(i, L)] = plsc.bitcast(
                    jnp.where(drop, 0, v), jnp.float32)
                return (i + L, c + plsc.all_reduce_population_count(eqm)[0])
            lax.while_loop(fx_cond, fx_body, (jnp.int32(0), jnp.int32(0)))

    pltpu.async_copy(in_hbm.at[base_row], row_a, sem_ai)
    for r in range(ROWS_PER_W):
        row_v, sem_i, sem_o = bufs[r % 2]

        @plsc.parallel_loop(0, NBUCKET * L, L, unroll=8)
        def zero_loop(i):
            hist_v[pl.ds(i, L)] = zeros

        with jax.named_scope("wait_in"):
            pltpu.make_async_copy(
                in_hbm.at[base_row + r], row_v, sem_i).wait()
        if r + 1 < ROWS_PER_W:
            nbuf, nsem_i, nsem_o = bufs[(r + 1) % 2]
            if r >= 1:
                pltpu.make_async_copy(
                    nbuf, out_hbm.at[base_row + r - 1], nsem_o).wait()
            pltpu.async_copy(in_hbm.at[base_row + r + 1], nbuf, nsem_i)
        process_row(row_v, base_row + r)
        pltpu.async_copy(row_v, out_hbm.at[base_row + r], sem_o)

    pltpu.make_async_copy(
        bufs[(ROWS_PER_W - 2) % 2][0],
        out_hbm.at[base_row + ROWS_PER_W - 2],
        bufs[(ROWS_PER_W - 2) % 2][2]).wait()
    pltpu.make_async_copy(
        bufs[(ROWS_PER_W - 1) % 2][0],
        out_hbm.at[base_row + ROWS_PER_W - 1],
        bufs[(ROWS_PER_W - 1) % 2][2]).wait()


_topk_mask = functools.partial(
    pl.kernel,
    out_type=jax.ShapeDtypeStruct((ROWS, COLS), jnp.float32),
    mesh=plsc.VectorSubcoreMesh(core_axis_name="c", subcore_axis_name="s"),
    scratch_types=[
        pltpu.VMEM((COLS,), jnp.float32),
        pltpu.VMEM((COLS,), jnp.float32),
        pltpu.VMEM((NBUCKET * L,), jnp.int32),
        pltpu.VMEM((CAND_ROWS * L,), jnp.int32),
        pltpu.SemaphoreType.DMA,
        pltpu.SemaphoreType.DMA,
        pltpu.SemaphoreType.DMA,
        pltpu.SemaphoreType.DMA,
    ],
    compiler_params=pltpu.CompilerParams(needs_layout_passes=False),
)(_sc_body)


@jax.jit
def kernel(input_):
    return _topk_mask(input_)


# speedup vs baseline: 1.2252x; 1.2252x over previous
import functools

import jax
import jax.numpy as jnp
from jax import lax
from jax.experimental import pallas as pl
from jax.experimental.pallas import tpu as pltpu
from jax.experimental.pallas import tpu_sc as plsc

ROWS = 128
COLS = 32768
K = 64
L = 16
BSHIFT = 21
NBUCKET = 1 << (31 - BSHIFT)
CAND_ROWS = 256
CAND_MASK = CAND_ROWS * L - 1
SIGN_MASK = 0x7FFFFFFF
EDGE_BITS = 0x40300000
HI_BITS = 0x7F7FFFFF
NC = 2
NS = 16
ROWS_PER_W = ROWS // (NC * NS)


def _sc_body(in_hbm, out_hbm, row_a, row_b, hist_v, cand_v,
             sem_ai, sem_bi, sem_ao, sem_bo):
    wid = lax.axis_index("s") * NC + lax.axis_index("c")
    lane = lax.iota(jnp.int32, L)
    ones = jnp.ones((L,), jnp.int32)
    zeros = jnp.zeros((L,), jnp.int32)
    base_row = wid * ROWS_PER_W
    bufs = [(row_a, sem_ai, sem_ao), (row_b, sem_bi, sem_bo)]

    def process_row(row_v):
        def fast_collect(i, cs):
            v = plsc.bitcast(row_v[pl.ds(i, L)], jnp.int32)
            ab = v & SIGN_MASK
            m = ab >= EDGE_BITS
            plsc.store_scatter(cand_v, [cs & CAND_MASK], ab, mask=m)
            return cs + jnp.where(m, L, 0)
        with jax.named_scope("collect"):
            cs = plsc.parallel_loop(0, COLS, L, unroll=8,
                                    carry=lane)(fast_collect)
        c_vec = (cs - lane) >> 4
        cmax = jnp.max(c_vec)
        total = jnp.sum(c_vec)
        fast_ok = jnp.logical_and(total >= K, cmax <= CAND_ROWS)

        def bs_and_ties(lo0, hi0, need, extra, c_vec_, max_c_, iters):
            def bs_loop(_, st):
                lo, hi = st
                mid = lo + ((hi - lo + 1) >> 1)

                def count_loop(j, acc):
                    cv = cand_v[pl.ds(j * L, L)]
                    pm = jnp.logical_and(cv >= mid, j < c_vec_)
                    return acc + jnp.where(pm, 1, 0)
                c = jnp.sum(lax.fori_loop(0, max_c_, count_loop, zeros))
                ok = c >= need
                return (jnp.where(ok, mid, lo), jnp.where(ok, hi, mid - 1))
            thr, _ = lax.fori_loop(0, iters, bs_loop, (lo0, hi0))

            def tie_loop(j, acc):
                cv = cand_v[pl.ds(j * L, L)]
                valid = j < c_vec_
                ge = jnp.logical_and(cv >= thr, valid)
                eq = jnp.logical_and(cv == thr, valid)
                return (acc[0] + jnp.where(ge, 1, 0),
                        acc[1] + jnp.where(eq, 1, 0))
            gev, eqv = lax.fori_loop(0, max_c_, tie_loop, (zeros, zeros))
            return thr, extra + jnp.sum(gev), jnp.sum(eqv)

        def fast_path(_):
            return bs_and_ties(jnp.int32(EDGE_BITS), jnp.int32(HI_BITS),
                               jnp.int32(K), jnp.int32(0), c_vec,
                               jnp.minimum(cmax, CAND_ROWS), 31)

        def slow_path(_):
            @plsc.parallel_loop(0, NBUCKET * L, L, unroll=8)
            def zero_loop(i):
                hist_v[pl.ds(i, L)] = zeros

            def hist_loop(i, mx):
                v = plsc.bitcast(row_v[pl.ds(i, L)], jnp.int32)
                ab = v & SIGN_MASK
                idx = ((ab >> BSHIFT) << 4) | lane
                plsc.addupdate_scatter(hist_v, [idx], ones)
                return jnp.maximum(mx, ab)
            mxv = plsc.parallel_loop(0, COLS, L, unroll=8,
                                     carry=zeros)(hist_loop)
            bstart = jnp.max(mxv) >> BSHIFT

            def scan_cond(st):
                b, cum, _ = st
                return jnp.logical_and(cum < K, b >= 0)

            def scan_body(st):
                b, cum, _ = st
                c = jnp.sum(hist_v[pl.ds(b * L, L)])
                return (b - 1, cum + c, c)
            bf, cum, lastc = lax.while_loop(
                scan_cond, scan_body, (bstart, jnp.int32(0), jnp.int32(0)))
            bucket = bf + 1
            need = K - (cum - lastc)
            cand_base = bucket << BSHIFT

            def re_collect(i, cs2):
                v = plsc.bitcast(row_v[pl.ds(i, L)], jnp.int32)
                ab = v & SIGN_MASK
                m = plsc.bitcast(ab - cand_base, jnp.uint32) < jnp.uint32(
                    1 << BSHIFT)
                plsc.store_scatter(cand_v, [cs2 & CAND_MASK], ab, mask=m)
                return cs2 + jnp.where(m, L, 0)
            cs2 = plsc.parallel_loop(0, COLS, L, unroll=8,
                                     carry=lane)(re_collect)
            c_vec2 = (cs2 - lane) >> 4
            return bs_and_ties(cand_base,
                               cand_base + (1 << BSHIFT) - 1,
                               need, K - need, c_vec2,
                               jnp.minimum(jnp.max(c_vec2), CAND_ROWS), 21)

        thr, n_ge, n_eq = lax.cond(fast_ok, fast_path, slow_path, 0)
        t_keep = K - (n_ge - n_eq)

        def mask_loop(i):
            v = plsc.bitcast(row_v[pl.ds(i, L)], jnp.int32)
            keep = (v & SIGN_MASK) >= thr
            row_v[pl.ds(i, L)] = plsc.bitcast(
                jnp.where(keep, v, 0), jnp.float32)
        with jax.named_scope("mask"):
            plsc.parallel_loop(0, COLS, L, unroll=8)(mask_loop)

        @pl.when(n_ge > K)
        def _fixup():
            def fx_cond(st):
                i, c = st
                return jnp.logical_and(i < COLS, c < n_eq)

            def fx_body(st):
                i, c = st
                v = plsc.bitcast(row_v[pl.ds(i, L)], jnp.int32)
                eqm = (v & SIGN_MASK) == thr
                rank = c + plsc.cumsum(jnp.where(eqm, 1, 0)) - 1
                drop = jnp.logical_and(eqm, rank >= t_keep)
                row_v[pl.ds(i, L)] = plsc.bitcast(
                    jnp.where(drop, 0, v), jnp.float32)
                return (i + L, c + plsc.all_reduce_population_count(eqm)[0])
            lax.while_loop(fx_cond, fx_body, (jnp.int32(0), jnp.int32(0)))

    pltpu.async_copy(in_hbm.at[base_row], row_a, sem_ai)
    for r in range(ROWS_PER_W):
        row_v, sem_i, sem_o = bufs[r % 2]
        with jax.named_scope("wait_in"):
            pltpu.make_async_copy(
                in_hbm.at[base_row + r], row_v, sem_i).wait()
        if r + 1 < ROWS_PER_W:
            nbuf, nsem_i, nsem_o = bufs[(r + 1) % 2]
            if r >= 1:
                pltpu.make_async_copy(
                    nbuf, out_hbm.at[base_row + r - 1], nsem_o).wait()
            pltpu.async_copy(in_hbm.at[base_row + r + 1], nbuf, nsem_i)
        process_row(row_v)
        pltpu.async_copy(row_v, out_hbm.at[base_row + r], sem_o)

    pltpu.make_async_copy(
        bufs[(ROWS_PER_W - 2) % 2][0],
        out_hbm.at[base_row + ROWS_PER_W - 2],
        bufs[(ROWS_PER_W - 2) % 2][2]).wait()
    pltpu.make_async_copy(
        bufs[(ROWS_PER_W - 1) % 2][0],
        out_hbm.at[base_row + ROWS_PER_W - 1],
        bufs[(ROWS_PER_W - 1) % 2][2]).wait()


_topk_mask = functools.partial(
    pl.kernel,
    out_type=jax.ShapeDtypeStruct((ROWS, COLS), jnp.float32),
    mesh=plsc.VectorSubcoreMesh(core_axis_name="c", subcore_axis_name="s"),
    scratch_types=[
        pltpu.VMEM((COLS,), jnp.float32),
        pltpu.VMEM((COLS,), jnp.float32),
        pltpu.VMEM((NBUCKET * L,), jnp.int32),
        pltpu.VMEM((CAND_ROWS * L,), jnp.int32),
        pltpu.SemaphoreType.DMA,
        pltpu.SemaphoreType.DMA,
        pltpu.SemaphoreType.DMA,
        pltpu.SemaphoreType.DMA,
    ],
    compiler_params=pltpu.CompilerParams(needs_layout_passes=False),
)(_sc_body)


@jax.jit
def kernel(input_):
    return _topk_mask(input_)


# speedup vs baseline: 1.4167x; 1.1562x over previous
import functools

import jax
import jax.numpy as jnp
from jax import lax
from jax.experimental import pallas as pl
from jax.experimental.pallas import tpu as pltpu
from jax.experimental.pallas import tpu_sc as plsc

ROWS = 128
COLS = 32768
K = 64
L = 16
BSHIFT = 21
NBUCKET = 1 << (31 - BSHIFT)
CAND_ROWS = 256
CAND_MASK = CAND_ROWS * L - 1
SIGN_MASK = 0x7FFFFFFF
EDGE_BITS = 0x40300000
HI_BITS = 0x7F7FFFFF
NC = 2
NS = 16
ROWS_PER_W = ROWS // (NC * NS)


def _sc_body(in_hbm, out_hbm, row_a, row_b, hist_v, cand_v,
             sem_ai, sem_bi, sem_ao, sem_bo):
    wid = lax.axis_index("s") * NC + lax.axis_index("c")
    lane = lax.iota(jnp.int32, L)
    ones = jnp.ones((L,), jnp.int32)
    zeros = jnp.zeros((L,), jnp.int32)
    base_row = wid * ROWS_PER_W
    bufs = [(row_a, sem_ai, sem_ao), (row_b, sem_bi, sem_bo)]

    def process_row(row_v):
        def fast_collect(i, cs):
            v = plsc.bitcast(row_v[pl.ds(i, L)], jnp.int32)
            ab = v & SIGN_MASK
            m = ab >= EDGE_BITS
            plsc.store_scatter(cand_v, [cs & CAND_MASK], ab, mask=m)
            return cs + jnp.where(m, L, 0)
        with jax.named_scope("collect"):
            cs = plsc.parallel_loop(0, COLS, L, unroll=8,
                                    carry=lane)(fast_collect)
        c_vec = (cs - lane) >> 4
        cmax = jnp.max(c_vec)
        cmin = jnp.min(c_vec)
        total = jnp.sum(c_vec)
        fast_ok = jnp.logical_and(total >= K, cmax <= CAND_ROWS)

        rows4 = (jnp.minimum(cmax, CAND_ROWS) + 3) >> 2

        def pad_loop(_, cs_p):
            m = cs_p < rows4 * (4 * L)
            plsc.store_scatter(cand_v, [cs_p & CAND_MASK], zeros, mask=m)
            return cs_p + jnp.where(m, L, 0)
        lax.fori_loop(0, (rows4 << 2) - cmin, pad_loop, cs)

        def bs_and_ties(lo0, hi0, need, extra, c_vec_, max_c_, iters):
            def bs_loop(_, st):
                lo, hi = st
                mid = lo + ((hi - lo + 1) >> 1)

                def count_loop(j, acc):
                    cv = cand_v[pl.ds(j * L, L)]
                    pm = jnp.logical_and(cv >= mid, j < c_vec_)
                    return acc + jnp.where(pm, 1, 0)
                c = jnp.sum(lax.fori_loop(0, max_c_, count_loop, zeros))
                ok = c >= need
                return (jnp.where(ok, mid, lo), jnp.where(ok, hi, mid - 1))
            thr, _ = lax.fori_loop(0, iters, bs_loop, (lo0, hi0))

            def tie_loop(j, acc):
                cv = cand_v[pl.ds(j * L, L)]
                valid = j < c_vec_
                ge = jnp.logical_and(cv >= thr, valid)
                eq = jnp.logical_and(cv == thr, valid)
                return (acc[0] + jnp.where(ge, 1, 0),
                        acc[1] + jnp.where(eq, 1, 0))
            gev, eqv = lax.fori_loop(0, max_c_, tie_loop, (zeros, zeros))
            return thr, extra + jnp.sum(gev), jnp.sum(eqv)

        def fast_path(_):
            def count4(mid):
                def count_loop(jj, acc):
                    b = jj << 6
                    a = acc + jnp.where(cand_v[pl.ds(b, L)] >= mid, 1, 0)
                    a = a + jnp.where(cand_v[pl.ds(b + L, L)] >= mid, 1, 0)
                    a = a + jnp.where(
                        cand_v[pl.ds(b + 2 * L, L)] >= mid, 1, 0)
                    a = a + jnp.where(
                        cand_v[pl.ds(b + 3 * L, L)] >= mid, 1, 0)
                    return a
                return jnp.sum(lax.fori_loop(0, rows4, count_loop, zeros))

            def bs_loop(_, st):
                lo, hi = st
                mid = lo + ((hi - lo + 1) >> 1)
                ok = count4(mid) >= K
                return (jnp.where(ok, mid, lo), jnp.where(ok, hi, mid - 1))
            thr, _ = lax.fori_loop(0, 31, bs_loop,
                                   (jnp.int32(EDGE_BITS),
                                    jnp.int32(HI_BITS)))

            def tie_loop(jj, acc):
                b = jj << 6
                ge, eq = acc
                for o in range(4):
                    cv = cand_v[pl.ds(b + o * L, L)]
                    ge = ge + jnp.where(cv >= thr, 1, 0)
                    eq = eq + jnp.where(cv == thr, 1, 0)
                return (ge, eq)
            gev, eqv = lax.fori_loop(0, rows4, tie_loop, (zeros, zeros))
            return thr, jnp.sum(gev), jnp.sum(eqv)

        def slow_path(_):
            @plsc.parallel_loop(0, NBUCKET * L, L, unroll=8)
            def zero_loop(i):
                hist_v[pl.ds(i, L)] = zeros

            def hist_loop(i, mx):
                v = plsc.bitcast(row_v[pl.ds(i, L)], jnp.int32)
                ab = v & SIGN_MASK
                idx = ((ab >> BSHIFT) << 4) | lane
                plsc.addupdate_scatter(hist_v, [idx], ones)
                return jnp.maximum(mx, ab)
            mxv = plsc.parallel_loop(0, COLS, L, unroll=8,
                                     carry=zeros)(hist_loop)
            bstart = jnp.max(mxv) >> BSHIFT

            def scan_cond(st):
                b, cum, _ = st
                return jnp.logical_and(cum < K, b >= 0)

            def scan_body(st):
                b, cum, _ = st
                c = jnp.sum(hist_v[pl.ds(b * L, L)])
                return (b - 1, cum + c, c)
            bf, cum, lastc = lax.while_loop(
                scan_cond, scan_body, (bstart, jnp.int32(0), jnp.int32(0)))
            bucket = bf + 1
            need = K - (cum - lastc)
            cand_base = bucket << BSHIFT

            def re_collect(i, cs2):
                v = plsc.bitcast(row_v[pl.ds(i, L)], jnp.int32)
                ab = v & SIGN_MASK
                m = plsc.bitcast(ab - cand_base, jnp.uint32) < jnp.uint32(
                    1 << BSHIFT)
                plsc.store_scatter(cand_v, [cs2 & CAND_MASK], ab, mask=m)
                return cs2 + jnp.where(m, L, 0)
            cs2 = plsc.parallel_loop(0, COLS, L, unroll=8,
                                     carry=lane)(re_collect)
            c_vec2 = (cs2 - lane) >> 4
            return bs_and_ties(cand_base,
                               cand_base + (1 << BSHIFT) - 1,
                               need, K - need, c_vec2,
                               jnp.minimum(jnp.max(c_vec2), CAND_ROWS), 21)

        thr, n_ge, n_eq = lax.cond(fast_ok, fast_path, slow_path, 0)
        t_keep = K - (n_ge - n_eq)

        def mask_loop(i):
            v = plsc.bitcast(row_v[pl.ds(i, L)], jnp.int32)
            keep = (v & SIGN_MASK) >= thr
            row_v[pl.ds(i, L)] = plsc.bitcast(
                jnp.where(keep, v, 0), jnp.float32)
        with jax.named_scope("mask"):
            plsc.parallel_loop(0, COLS, L, unroll=8)(mask_loop)

        @pl.when(n_ge > K)
        def _fixup():
            def fx_cond(st):
                i, c = st
                return jnp.logical_and(i < COLS, c < n_eq)

            def fx_body(st):
                i, c = st
                v = plsc.bitcast(row_v[pl.ds(i, L)], jnp.int32)
                eqm = (v & SIGN_MASK) == thr
                rank = c + plsc.cumsum(jnp.where(eqm, 1, 0)) - 1
                drop = jnp.logical_and(eqm, rank >= t_keep)
                row_v[pl.ds(i, L)] = plsc.bitcast(
                    jnp.where(drop, 0, v), jnp.float32)
                return (i + L, c + plsc.all_reduce_population_count(eqm)[0])
            lax.while_loop(fx_cond, fx_body, (jnp.int32(0), jnp.int32(0)))

    pltpu.async_copy(in_hbm.at[base_row], row_a, sem_ai)
    for r in range(ROWS_PER_W):
        row_v, sem_i, sem_o = bufs[r % 2]
        with jax.named_scope("wait_in"):
            pltpu.make_async_copy(
                in_hbm.at[base_row + r], row_v, sem_i).wait()
        if r + 1 < ROWS_PER_W:
            nbuf, nsem_i, nsem_o = bufs[(r + 1) % 2]
            if r >= 1:
                pltpu.make_async_copy(
                    nbuf, out_hbm.at[base_row + r - 1], nsem_o).wait()
            pltpu.async_copy(in_hbm.at[base_row + r + 1], nbuf, nsem_i)
        process_row(row_v)
        pltpu.async_copy(row_v, out_hbm.at[base_row + r], sem_o)

    pltpu.make_async_copy(
        bufs[(ROWS_PER_W - 2) % 2][0],
        out_hbm.at[base_row + ROWS_PER_W - 2],
        bufs[(ROWS_PER_W - 2) % 2][2]).wait()
    pltpu.make_async_copy(
        bufs[(ROWS_PER_W - 1) % 2][0],
        out_hbm.at[base_row + ROWS_PER_W - 1],
        bufs[(ROWS_PER_W - 1) % 2][2]).wait()


_topk_mask = functools.partial(
    pl.kernel,
    out_type=jax.ShapeDtypeStruct((ROWS, COLS), jnp.float32),
    mesh=plsc.VectorSubcoreMesh(core_axis_name="c", subcore_axis_name="s"),
    scratch_types=[
        pltpu.VMEM((COLS,), jnp.float32),
        pltpu.VMEM((COLS,), jnp.float32),
        pltpu.VMEM((NBUCKET * L,), jnp.int32),
        pltpu.VMEM((CAND_ROWS * L,), jnp.int32),
        pltpu.SemaphoreType.DMA,
        pltpu.SemaphoreType.DMA,
        pltpu.SemaphoreType.DMA,
        pltpu.SemaphoreType.DMA,
    ],
    compiler_params=pltpu.CompilerParams(needs_layout_passes=False),
)(_sc_body)


@jax.jit
def kernel(input_):
    return _topk_mask(input_)


# speedup vs baseline: 1.4378x; 1.0150x over previous
import functools

import jax
import jax.numpy as jnp
from jax import lax
from jax.experimental import pallas as pl
from jax.experimental.pallas import tpu as pltpu
from jax.experimental.pallas import tpu_sc as plsc

ROWS = 128
COLS = 32768
K = 64
L = 16
BSHIFT = 21
NBUCKET = 1 << (31 - BSHIFT)
CAND_ROWS = 256
CAND_MASK = CAND_ROWS * L - 1
SIGN_MASK = 0x7FFFFFFF
EDGE_BITS = 0x40380000
NC = 2
NS = 16
ROWS_PER_W = ROWS // (NC * NS)


def _sc_body(in_hbm, out_hbm, row_a, row_b, hist_v, cand_v,
             sem_ai, sem_bi, sem_ao, sem_bo):
    wid = lax.axis_index("s") * NC + lax.axis_index("c")
    lane = lax.iota(jnp.int32, L)
    ones = jnp.ones((L,), jnp.int32)
    zeros = jnp.zeros((L,), jnp.int32)
    base_row = wid * ROWS_PER_W
    bufs = [(row_a, sem_ai, sem_ao), (row_b, sem_bi, sem_bo)]

    def process_row(row_v):
        def fast_collect(i, cs):
            v = plsc.bitcast(row_v[pl.ds(i, L)], jnp.int32)
            ab = v & SIGN_MASK
            m = ab >= EDGE_BITS
            plsc.store_scatter(cand_v, [cs & CAND_MASK], ab, mask=m)
            return cs + jnp.where(m, L, 0)
        with jax.named_scope("collect"):
            cs = plsc.parallel_loop(0, COLS, L, unroll=8,
                                    carry=lane)(fast_collect)
        c_vec = (cs - lane) >> 4
        cmax = jnp.max(c_vec)
        cmin = jnp.min(c_vec)
        total = jnp.sum(c_vec)
        fast_ok = jnp.logical_and(total >= K, cmax <= CAND_ROWS)

        rows4 = (jnp.minimum(cmax, CAND_ROWS) + 3) >> 2

        def pad_loop(_, cs_p):
            m = cs_p < rows4 * (4 * L)
            plsc.store_scatter(cand_v, [cs_p & CAND_MASK], zeros, mask=m)
            return cs_p + jnp.where(m, L, 0)
        lax.fori_loop(0, (rows4 << 2) - cmin, pad_loop, cs)

        def bs_and_ties(lo0, hi0, need, extra, c_vec_, max_c_, iters):
            def bs_loop(_, st):
                lo, hi = st
                mid = lo + ((hi - lo + 1) >> 1)

                def count_loop(j, acc):
                    cv = cand_v[pl.ds(j * L, L)]
                    pm = jnp.logical_and(cv >= mid, j < c_vec_)
                    return acc + jnp.where(pm, 1, 0)
                c = jnp.sum(lax.fori_loop(0, max_c_, count_loop, zeros))
                ok = c >= need
                return (jnp.where(ok, mid, lo), jnp.where(ok, hi, mid - 1))
            thr, _ = lax.fori_loop(0, iters, bs_loop, (lo0, hi0))

            def tie_loop(j, acc):
                cv = cand_v[pl.ds(j * L, L)]
                valid = j < c_vec_
                ge = jnp.logical_and(cv >= thr, valid)
                eq = jnp.logical_and(cv == thr, valid)
                return (acc[0] + jnp.where(ge, 1, 0),
                        acc[1] + jnp.where(eq, 1, 0))
            gev, eqv = lax.fori_loop(0, max_c_, tie_loop, (zeros, zeros))
            return thr, extra + jnp.sum(gev), jnp.sum(eqv)

        def fast_path(_):
            def count4(mid):
                def count_loop(jj, acc):
                    b = jj << 6
                    a = acc + jnp.where(cand_v[pl.ds(b, L)] >= mid, 1, 0)
                    a = a + jnp.where(cand_v[pl.ds(b + L, L)] >= mid, 1, 0)
                    a = a + jnp.where(
                        cand_v[pl.ds(b + 2 * L, L)] >= mid, 1, 0)
                    a = a + jnp.where(
                        cand_v[pl.ds(b + 3 * L, L)] >= mid, 1, 0)
                    return a
                return jnp.sum(lax.fori_loop(0, rows4, count_loop, zeros))

            def max_loop(jj, acc):
                b = jj << 6
                a = jnp.maximum(acc, cand_v[pl.ds(b, L)])
                a = jnp.maximum(a, cand_v[pl.ds(b + L, L)])
                a = jnp.maximum(a, cand_v[pl.ds(b + 2 * L, L)])
                return jnp.maximum(a, cand_v[pl.ds(b + 3 * L, L)])
            hi0 = jnp.max(lax.fori_loop(0, rows4, max_loop, zeros))

            def bs_cond(st):
                lo, hi = st
                return lo < hi

            def bs_loop(st):
                lo, hi = st
                mid = lo + ((hi - lo + 1) >> 1)
                ok = count4(mid) >= K
                return (jnp.where(ok, mid, lo), jnp.where(ok, hi, mid - 1))
            thr, _ = lax.while_loop(bs_cond, bs_loop,
                                    (jnp.int32(EDGE_BITS), hi0))

            def tie_loop(jj, acc):
                b = jj << 6
                ge, eq = acc
                for o in range(4):
                    cv = cand_v[pl.ds(b + o * L, L)]
                    ge = ge + jnp.where(cv >= thr, 1, 0)
                    eq = eq + jnp.where(cv == thr, 1, 0)
                return (ge, eq)
            gev, eqv = lax.fori_loop(0, rows4, tie_loop, (zeros, zeros))
            return thr, jnp.sum(gev), jnp.sum(eqv)

        def slow_path(_):
            @plsc.parallel_loop(0, NBUCKET * L, L, unroll=8)
            def zero_loop(i):
                hist_v[pl.ds(i, L)] = zeros

            def hist_loop(i, mx):
                v = plsc.bitcast(row_v[pl.ds(i, L)], jnp.int32)
                ab = v & SIGN_MASK
                idx = ((ab >> BSHIFT) << 4) | lane
                plsc.addupdate_scatter(hist_v, [idx], ones)
                return jnp.maximum(mx, ab)
            mxv = plsc.parallel_loop(0, COLS, L, unroll=8,
                                     carry=zeros)(hist_loop)
            bstart = jnp.max(mxv) >> BSHIFT

            def scan_cond(st):
                b, cum, _ = st
                return jnp.logical_and(cum < K, b >= 0)

            def scan_body(st):
                b, cum, _ = st
                c = jnp.sum(hist_v[pl.ds(b * L, L)])
                return (b - 1, cum + c, c)
            bf, cum, lastc = lax.while_loop(
                scan_cond, scan_body, (bstart, jnp.int32(0), jnp.int32(0)))
            bucket = bf + 1
            need = K - (cum - lastc)
            cand_base = bucket << BSHIFT

            def re_collect(i, cs2):
                v = plsc.bitcast(row_v[pl.ds(i, L)], jnp.int32)
                ab = v & SIGN_MASK
                m = plsc.bitcast(ab - cand_base, jnp.uint32) < jnp.uint32(
                    1 << BSHIFT)
                plsc.store_scatter(cand_v, [cs2 & CAND_MASK], ab, mask=m)
                return cs2 + jnp.where(m, L, 0)
            cs2 = plsc.parallel_loop(0, COLS, L, unroll=8,
                                     carry=lane)(re_collect)
            c_vec2 = (cs2 - lane) >> 4
            return bs_and_ties(cand_base,
                               cand_base + (1 << BSHIFT) - 1,
                               need, K - need, c_vec2,
                               jnp.minimum(jnp.max(c_vec2), CAND_ROWS), 21)

        thr, n_ge, n_eq = lax.cond(fast_ok, fast_path, slow_path, 0)
        t_keep = K - (n_ge - n_eq)

        def mask_loop(i):
            v = plsc.bitcast(row_v[pl.ds(i, L)], jnp.int32)
            keep = (v & SIGN_MASK) >= thr
            row_v[pl.ds(i, L)] = plsc.bitcast(
                jnp.where(keep, v, 0), jnp.float32)
        with jax.named_scope("mask"):
            plsc.parallel_loop(0, COLS, L, unroll=8)(mask_loop)

        @pl.when(n_ge > K)
        def _fixup():
            def fx_cond(st):
                i, c = st
                return jnp.logical_and(i < COLS, c < n_eq)

            def fx_body(st):
                i, c = st
                v = plsc.bitcast(row_v[pl.ds(i, L)], jnp.int32)
                eqm = (v & SIGN_MASK) == thr
                rank = c + plsc.cumsum(jnp.where(eqm, 1, 0)) - 1
                drop = jnp.logical_and(eqm, rank >= t_keep)
                row_v[pl.ds(i, L)] = plsc.bitcast(
                    jnp.where(drop, 0, v), jnp.float32)
                return (i + L, c + plsc.all_reduce_population_count(eqm)[0])
            lax.while_loop(fx_cond, fx_body, (jnp.int32(0), jnp.int32(0)))

    pltpu.async_copy(in_hbm.at[base_row], row_a, sem_ai)
    for r in range(ROWS_PER_W):
        row_v, sem_i, sem_o = bufs[r % 2]
        with jax.named_scope("wait_in"):
            pltpu.make_async_copy(
                in_hbm.at[base_row + r], row_v, sem_i).wait()
        if r + 1 < ROWS_PER_W:
            nbuf, nsem_i, nsem_o = bufs[(r + 1) % 2]
            if r >= 1:
                pltpu.make_async_copy(
                    nbuf, out_hbm.at[base_row + r - 1], nsem_o).wait()
            pltpu.async_copy(in_hbm.at[base_row + r + 1], nbuf, nsem_i)
        process_row(row_v)
        pltpu.async_copy(row_v, out_hbm.at[base_row + r], sem_o)

    pltpu.make_async_copy(
        bufs[(ROWS_PER_W - 2) % 2][0],
        out_hbm.at[base_row + ROWS_PER_W - 2],
        bufs[(ROWS_PER_W - 2) % 2][2]).wait()
    pltpu.make_async_copy(
        bufs[(ROWS_PER_W - 1) % 2][0],
        out_hbm.at[base_row + ROWS_PER_W - 1],
        bufs[(ROWS_PER_W - 1) % 2][2]).wait()


_topk_mask = functools.partial(
    pl.kernel,
    out_type=jax.ShapeDtypeStruct((ROWS, COLS), jnp.float32),
    mesh=plsc.VectorSubcoreMesh(core_axis_name="c", subcore_axis_name="s"),
    scratch_types=[
        pltpu.VMEM((COLS,), jnp.float32),
        pltpu.VMEM((COLS,), jnp.float32),
        pltpu.VMEM((NBUCKET * L,), jnp.int32),
        pltpu.VMEM((CAND_ROWS * L,), jnp.int32),
        pltpu.SemaphoreType.DMA,
        pltpu.SemaphoreType.DMA,
        pltpu.SemaphoreType.DMA,
        pltpu.SemaphoreType.DMA,
    ],
    compiler_params=pltpu.CompilerParams(needs_layout_passes=False),
)(_sc_body)


@jax.jit
def kernel(input_):
    return _topk_mask(input_)


# speedup vs baseline: 1.4484x; 1.0073x over previous
import functools

import jax
import jax.numpy as jnp
from jax import lax
from jax.experimental import pallas as pl
from jax.experimental.pallas import tpu as pltpu
from jax.experimental.pallas import tpu_sc as plsc

ROWS = 128
COLS = 32768
K = 64
L = 16
BSHIFT = 21
NBUCKET = 1 << (31 - BSHIFT)
CAND_ROWS = 256
CAND_MASK = CAND_ROWS * L - 1
SIGN_MASK = 0x7FFFFFFF
EDGE_BITS = 0x40380000
NC = 2
NS = 16
ROWS_PER_W = ROWS // (NC * NS)


def _sc_body(in_hbm, out_hbm, row_a, row_b, hist_v, cand_v,
             sem_ai, sem_bi, sem_ao, sem_bo):
    wid = lax.axis_index("s") * NC + lax.axis_index("c")
    lane = lax.iota(jnp.int32, L)
    ones = jnp.ones((L,), jnp.int32)
    zeros = jnp.zeros((L,), jnp.int32)
    base_row = wid * ROWS_PER_W
    bufs = [(row_a, sem_ai, sem_ao), (row_b, sem_bi, sem_bo)]

    def process_row(row_v):
        def fast_collect(i, cs):
            v = plsc.bitcast(row_v[pl.ds(i, L)], jnp.int32)
            ab = v & SIGN_MASK
            m = ab >= EDGE_BITS
            plsc.store_scatter(cand_v, [cs & CAND_MASK], ab, mask=m)
            return cs + jnp.where(m, L, 0)
        cs = plsc.parallel_loop(0, COLS, L, unroll=8,
                                carry=lane)(fast_collect)
        c_vec = (cs - lane) >> 4
        cmax = jnp.max(c_vec)
        cmin = jnp.min(c_vec)
        total = jnp.sum(c_vec)
        fast_ok = jnp.logical_and(total >= K, cmax <= CAND_ROWS)

        rows4 = (jnp.minimum(cmax, CAND_ROWS) + 3) >> 2

        def pad_loop(_, cs_p):
            m = cs_p < rows4 * (4 * L)
            plsc.store_scatter(cand_v, [cs_p & CAND_MASK], zeros, mask=m)
            return cs_p + jnp.where(m, L, 0)
        lax.fori_loop(0, (rows4 << 2) - cmin, pad_loop, cs)

        def bs_and_ties(lo0, hi0, need, extra, c_vec_, max_c_, iters):
            def bs_loop(_, st):
                lo, hi = st
                mid = lo + ((hi - lo + 1) >> 1)

                def count_loop(j, acc):
                    cv = cand_v[pl.ds(j * L, L)]
                    pm = jnp.logical_and(cv >= mid, j < c_vec_)
                    return acc + jnp.where(pm, 1, 0)
                c = jnp.sum(lax.fori_loop(0, max_c_, count_loop, zeros))
                ok = c >= need
                return (jnp.where(ok, mid, lo), jnp.where(ok, hi, mid - 1))
            thr, _ = lax.fori_loop(0, iters, bs_loop, (lo0, hi0))

            def tie_loop(j, acc):
                cv = cand_v[pl.ds(j * L, L)]
                valid = j < c_vec_
                ge = jnp.logical_and(cv >= thr, valid)
                eq = jnp.logical_and(cv == thr, valid)
                return (acc[0] + jnp.where(ge, 1, 0),
                        acc[1] + jnp.where(eq, 1, 0))
            gev, eqv = lax.fori_loop(0, max_c_, tie_loop, (zeros, zeros))
            return thr, extra + jnp.sum(gev), jnp.sum(eqv)

        def fast_path(_):
            def count4(mid):
                def count_loop(jj, acc):
                    b = jj << 6
                    a = acc + jnp.where(cand_v[pl.ds(b, L)] >= mid, 1, 0)
                    a = a + jnp.where(cand_v[pl.ds(b + L, L)] >= mid, 1, 0)
                    a = a + jnp.where(
                        cand_v[pl.ds(b + 2 * L, L)] >= mid, 1, 0)
                    a = a + jnp.where(
                        cand_v[pl.ds(b + 3 * L, L)] >= mid, 1, 0)
                    return a
                return jnp.sum(lax.fori_loop(0, rows4, count_loop, zeros))

            def max_loop(jj, acc):
                b = jj << 6
                a = jnp.maximum(acc, cand_v[pl.ds(b, L)])
                a = jnp.maximum(a, cand_v[pl.ds(b + L, L)])
                a = jnp.maximum(a, cand_v[pl.ds(b + 2 * L, L)])
                return jnp.maximum(a, cand_v[pl.ds(b + 3 * L, L)])
            hi0 = jnp.max(lax.fori_loop(0, rows4, max_loop, zeros))

            def bs_cond(st):
                lo, hi = st
                return lo < hi

            def bs_loop(st):
                lo, hi = st
                mid = lo + ((hi - lo + 1) >> 1)
                ok = count4(mid) >= K
                return (jnp.where(ok, mid, lo), jnp.where(ok, hi, mid - 1))
            thr, _ = lax.while_loop(bs_cond, bs_loop,
                                    (jnp.int32(EDGE_BITS), hi0))

            def tie_loop(jj, acc):
                b = jj << 6
                ge, eq = acc
                for o in range(4):
                    cv = cand_v[pl.ds(b + o * L, L)]
                    ge = ge + jnp.where(cv >= thr, 1, 0)
                    eq = eq + jnp.where(cv == thr, 1, 0)
                return (ge, eq)
            gev, eqv = lax.fori_loop(0, rows4, tie_loop, (zeros, zeros))
            return thr, jnp.sum(gev), jnp.sum(eqv)

        def slow_path(_):
            @plsc.parallel_loop(0, NBUCKET * L, L, unroll=8)
            def zero_loop(i):
                hist_v[pl.ds(i, L)] = zeros

            def hist_loop(i, mx):
                v = plsc.bitcast(row_v[pl.ds(i, L)], jnp.int32)
                ab = v & SIGN_MASK
                idx = ((ab >> BSHIFT) << 4) | lane
                plsc.addupdate_scatter(hist_v, [idx], ones)
                return jnp.maximum(mx, ab)
            mxv = plsc.parallel_loop(0, COLS, L, unroll=8,
                                     carry=zeros)(hist_loop)
            bstart = jnp.max(mxv) >> BSHIFT

            def scan_cond(st):
                b, cum, _ = st
                return jnp.logical_and(cum < K, b >= 0)

            def scan_body(st):
                b, cum, _ = st
                c = jnp.sum(hist_v[pl.ds(b * L, L)])
                return (b - 1, cum + c, c)
            bf, cum, lastc = lax.while_loop(
                scan_cond, scan_body, (bstart, jnp.int32(0), jnp.int32(0)))
            bucket = bf + 1
            need = K - (cum - lastc)
            cand_base = bucket << BSHIFT

            def re_collect(i, cs2):
                v = plsc.bitcast(row_v[pl.ds(i, L)], jnp.int32)
                ab = v & SIGN_MASK
                m = plsc.bitcast(ab - cand_base, jnp.uint32) < jnp.uint32(
                    1 << BSHIFT)
                plsc.store_scatter(cand_v, [cs2 & CAND_MASK], ab, mask=m)
                return cs2 + jnp.where(m, L, 0)
            cs2 = plsc.parallel_loop(0, COLS, L, unroll=8,
                                     carry=lane)(re_collect)
            c_vec2 = (cs2 - lane) >> 4
            return bs_and_ties(cand_base,
                               cand_base + (1 << BSHIFT) - 1,
                               need, K - need, c_vec2,
                               jnp.minimum(jnp.max(c_vec2), CAND_ROWS), 21)

        thr, n_ge, n_eq = lax.cond(fast_ok, fast_path, slow_path, 0)
        t_keep = K - (n_ge - n_eq)

        def mask_loop(i):
            v = plsc.bitcast(row_v[pl.ds(i, L)], jnp.int32)
            keep = (v & SIGN_MASK) >= thr
            row_v[pl.ds(i, L)] = plsc.bitcast(
                jnp.where(keep, v, 0), jnp.float32)
        plsc.parallel_loop(0, COLS, L, unroll=8)(mask_loop)

        @pl.when(n_ge > K)
        def _fixup():
            def fx_cond(st):
                i, c = st
                return jnp.logical_and(i < COLS, c < n_eq)

            def fx_body(st):
                i, c = st
                v = plsc.bitcast(row_v[pl.ds(i, L)], jnp.int32)
                eqm = (v & SIGN_MASK) == thr
                rank = c + plsc.cumsum(jnp.where(eqm, 1, 0)) - 1
                drop = jnp.logical_and(eqm, rank >= t_keep)
                row_v[pl.ds(i, L)] = plsc.bitcast(
                    jnp.where(drop, 0, v), jnp.float32)
                return (i + L, c + plsc.all_reduce_population_count(eqm)[0])
            lax.while_loop(fx_cond, fx_body, (jnp.int32(0), jnp.int32(0)))

    pltpu.async_copy(in_hbm.at[base_row], row_a, sem_ai)
    for r in range(ROWS_PER_W):
        row_v, sem_i, sem_o = bufs[r % 2]
        pltpu.make_async_copy(in_hbm.at[base_row + r], row_v, sem_i).wait()
        if r + 1 < ROWS_PER_W:
            nbuf, nsem_i, nsem_o = bufs[(r + 1) % 2]
            if r >= 1:
                pltpu.make_async_copy(
                    nbuf, out_hbm.at[base_row + r - 1], nsem_o).wait()
            pltpu.async_copy(in_hbm.at[base_row + r + 1], nbuf, nsem_i)
        process_row(row_v)
        pltpu.async_copy(row_v, out_hbm.at[base_row + r], sem_o)

    pltpu.make_async_copy(
        bufs[(ROWS_PER_W - 2) % 2][0],
        out_hbm.at[base_row + ROWS_PER_W - 2],
        bufs[(ROWS_PER_W - 2) % 2][2]).wait()
    pltpu.make_async_copy(
        bufs[(ROWS_PER_W - 1) % 2][0],
        out_hbm.at[base_row + ROWS_PER_W - 1],
        bufs[(ROWS_PER_W - 1) % 2][2]).wait()


_topk_mask = functools.partial(
    pl.kernel,
    out_type=jax.ShapeDtypeStruct((ROWS, COLS), jnp.float32),
    mesh=plsc.VectorSubcoreMesh(core_axis_name="c", subcore_axis_name="s"),
    scratch_types=[
        pltpu.VMEM((COLS,), jnp.float32),
        pltpu.VMEM((COLS,), jnp.float32),
        pltpu.VMEM((NBUCKET * L,), jnp.int32),
        pltpu.VMEM((CAND_ROWS * L,), jnp.int32),
        pltpu.SemaphoreType.DMA,
        pltpu.SemaphoreType.DMA,
        pltpu.SemaphoreType.DMA,
        pltpu.SemaphoreType.DMA,
    ],
    compiler_params=pltpu.CompilerParams(needs_layout_passes=False),
)(_sc_body)


@jax.jit
def kernel(input_):
    return _topk_mask(input_)
